# baseline (device time: 15331 ns/iter reference)
import jax
import jax.numpy as jnp
from jax import lax
from jax.experimental import pallas as pl
from jax.experimental.pallas import tpu as pltpu

N_DEV = 4
B = 2
SQ = 128
D = 512
HQ = 8
HKV = 2
DH = 64
GROUP = HQ // HKV
SCALE = 0.125


def kernel(x, Wq, Wo, K_ext, V_ext):
    skv_loc = K_ext.shape[1]
    K2 = K_ext.reshape(B, skv_loc, HKV * DH).astype(jnp.bfloat16)
    V2 = V_ext.reshape(B, skv_loc, HKV * DH).astype(jnp.bfloat16)

    def body(x_hbm, wq_hbm, wo_hbm, k_hbm, v_hbm, out_ref,
             x_scr, wq_scr, wo_scr,
             kvbuf, load_sems, send_sems, recv_sems):
        my = lax.axis_index("i")

        loads = []
        for i, (src, dst) in enumerate(
            [(k_hbm, kvbuf.at[0, 0]), (v_hbm, kvbuf.at[0, 1]),
             (x_hbm, x_scr), (wq_hbm, wq_scr), (wo_hbm, wo_scr)]
        ):
            cp = pltpu.make_async_copy(src, dst, load_sems.at[i])
            cp.start()
            loads.append(cp)

        barrier_sem = pltpu.get_barrier_semaphore()
        for d in range(1, N_DEV):
            pl.semaphore_signal(
                barrier_sem, inc=1,
                device_id=((my + d) % N_DEV,),
                device_id_type=pl.DeviceIdType.MESH,
            )
        loads[0].wait()
        loads[1].wait()
        pl.semaphore_wait(barrier_sem, N_DEV - 1)

        sends = []
        for d in range(1, N_DEV):
            r = pltpu.make_async_remote_copy(
                src_ref=kvbuf.at[0], dst_ref=kvbuf.at[N_DEV - d],
                send_sem=send_sems.at[d - 1], recv_sem=recv_sems.at[N_DEV - d],
                device_id=((my + d) % N_DEV,),
                device_id_type=pl.DeviceIdType.MESH,
            )
            r.start()
            sends.append(r)

        loads[2].wait()
        loads[3].wait()
        wq = (wq_scr[...] * SCALE).astype(jnp.bfloat16)
        q_stacks = []
        for b in range(B):
            q_b = lax.dot_general(
                x_scr[b].astype(jnp.bfloat16), wq, (((1,), (0,)), ((), ())),
                preferred_element_type=jnp.float32,
            ).astype(jnp.bfloat16)
            for g in range(HKV):
                q_stacks.append(jnp.concatenate(
                    [q_b[:, (g * GROUP + j) * DH:(g * GROUP + j + 1) * DH]
                     for j in range(GROUP)], axis=0))

        ones_blk = jnp.ones((skv_loc, DH), jnp.bfloat16)

        def accumulate(slot, acc):
            out = []
            for b in range(B):
                k_c = kvbuf[slot, 0, b]
                v_c = kvbuf[slot, 1, b]
                for g in range(HKV):
                    k_g = k_c[:, g * DH:(g + 1) * DH]
                    v_aug = jnp.concatenate(
                        [v_c[:, g * DH:(g + 1) * DH], ones_blk], axis=1
                    )
                    p = jnp.exp(lax.dot_general(
                        q_stacks[b * HKV + g], k_g, (((1,), (1,)), ((), ())),
                        preferred_element_type=jnp.float32,
                    ).astype(jnp.bfloat16))
                    o = lax.dot_general(
                        p, v_aug, (((1,), (0,)), ((), ())),
                        preferred_element_type=jnp.float32,
                    )
                    i = b * HKV + g
                    out.append(o if acc is None else acc[i] + o)
            return out

        acc = accumulate(0, None)
        for slot in (1, 3, 2):
            pltpu.make_async_remote_copy(
                src_ref=kvbuf.at[slot], dst_ref=kvbuf.at[slot],
                send_sem=send_sems.at[0], recv_sem=recv_sems.at[slot],
                device_id=(my,), device_id_type=pl.DeviceIdType.MESH,
            ).wait_recv()
            acc = accumulate(slot, acc)

        loads[4].wait()
        wo = wo_scr[...].astype(jnp.bfloat16)
        for b in range(B):
            attn_b = jnp.concatenate(
                [(acc[b * HKV + g][j * SQ:(j + 1) * SQ, :DH] /
                  acc[b * HKV + g][j * SQ:(j + 1) * SQ, DH:DH + 1])
                 for g in range(HKV) for j in range(GROUP)], axis=1
            ).astype(jnp.bfloat16)
            out_ref[b] = lax.dot_general(
                attn_b, wo, (((1,), (0,)), ((), ())),
                preferred_element_type=jnp.float32,
            )

        for r in sends:
            r.wait_send()

    return pl.pallas_call(
        body,
        out_shape=jax.ShapeDtypeStruct((B, SQ, D), jnp.float32),
        in_specs=[pl.BlockSpec(memory_space=pltpu.HBM)] * 5,
        out_specs=pl.BlockSpec(memory_space=pltpu.VMEM),
        scratch_shapes=[
            pltpu.VMEM((B, SQ, D), jnp.float32),
            pltpu.VMEM((D, D), jnp.float32),
            pltpu.VMEM((D, D), jnp.float32),
            pltpu.VMEM((N_DEV, 2, B, skv_loc, HKV * DH), jnp.bfloat16),
            pltpu.SemaphoreType.DMA((5,)),
            pltpu.SemaphoreType.DMA((N_DEV - 1,)),
            pltpu.SemaphoreType.DMA((N_DEV,)),
        ],
        compiler_params=pltpu.CompilerParams(collective_id=0),
    )(x, Wq, Wo, K2, V2)


# device time: 15078 ns/iter; 1.0168x vs baseline; 1.0168x over previous
import jax
import jax.numpy as jnp
from jax import lax
from jax.experimental import pallas as pl
from jax.experimental.pallas import tpu as pltpu

N_DEV = 4
B = 2
SQ = 128
D = 512
HQ = 8
HKV = 2
DH = 64
GROUP = HQ // HKV
SCALE = 0.125


def kernel(x, Wq, Wo, K_ext, V_ext):
    skv_loc = K_ext.shape[1]
    K2 = K_ext.reshape(B, skv_loc, HKV * DH).astype(jnp.bfloat16)
    V2 = V_ext.reshape(B, skv_loc, HKV * DH).astype(jnp.bfloat16)
    x2 = x.astype(jnp.bfloat16)
    Wq2 = (Wq * SCALE).astype(jnp.bfloat16)
    Wo2 = Wo.astype(jnp.bfloat16)

    def body(x_hbm, wq_hbm, wo_hbm, k_hbm, v_hbm, out_ref,
             x_scr, wq_scr, wo_scr,
             kvbuf, load_sems, send_sems, recv_sems):
        my = lax.axis_index("i")

        loads = []
        for i, (src, dst) in enumerate(
            [(k_hbm, kvbuf.at[0, 0]), (v_hbm, kvbuf.at[0, 1]),
             (x_hbm, x_scr), (wq_hbm, wq_scr), (wo_hbm, wo_scr)]
        ):
            cp = pltpu.make_async_copy(src, dst, load_sems.at[i])
            cp.start()
            loads.append(cp)

        barrier_sem = pltpu.get_barrier_semaphore()
        for d in range(1, N_DEV):
            pl.semaphore_signal(
                barrier_sem, inc=1,
                device_id=((my + d) % N_DEV,),
                device_id_type=pl.DeviceIdType.MESH,
            )
        loads[0].wait()
        loads[1].wait()
        pl.semaphore_wait(barrier_sem, N_DEV - 1)

        sends = []
        for d in range(1, N_DEV):
            for h in range(2):
                rows = pl.ds(h * (skv_loc // 2), skv_loc // 2)
                r = pltpu.make_async_remote_copy(
                    src_ref=kvbuf.at[0, :, :, rows, :],
                    dst_ref=kvbuf.at[N_DEV - d, :, :, rows, :],
                    send_sem=send_sems.at[d - 1, h],
                    recv_sem=recv_sems.at[N_DEV - d, h],
                    device_id=((my + d) % N_DEV,),
                    device_id_type=pl.DeviceIdType.MESH,
                )
                r.start()
                sends.append(r)

        loads[2].wait()
        loads[3].wait()
        wq = wq_scr[...]
        q_stacks = []
        for b in range(B):
            q_b = lax.dot_general(
                x_scr[b], wq, (((1,), (0,)), ((), ())),
                preferred_element_type=jnp.float32,
            ).astype(jnp.bfloat16)
            for g in range(HKV):
                q_stacks.append(jnp.concatenate(
                    [q_b[:, (g * GROUP + j) * DH:(g * GROUP + j + 1) * DH]
                     for j in range(GROUP)], axis=0))

        half = skv_loc // 2
        ones_blk = jnp.ones((half, DH), jnp.bfloat16)

        def accumulate(slot, h, acc):
            out = []
            for b in range(B):
                k_c = kvbuf[slot, 0, b, h * half:(h + 1) * half]
                v_c = kvbuf[slot, 1, b, h * half:(h + 1) * half]
                for g in range(HKV):
                    k_g = k_c[:, g * DH:(g + 1) * DH]
                    v_aug = jnp.concatenate(
                        [v_c[:, g * DH:(g + 1) * DH], ones_blk], axis=1
                    )
                    p = jnp.exp(lax.dot_general(
                        q_stacks[b * HKV + g], k_g, (((1,), (1,)), ((), ())),
                        preferred_element_type=jnp.float32,
                    ).astype(jnp.bfloat16))
                    o = lax.dot_general(
                        p, v_aug, (((1,), (0,)), ((), ())),
                        preferred_element_type=jnp.float32,
                    )
                    i = b * HKV + g
                    out.append(o if acc is None else acc[i] + o)
            return out

        acc = accumulate(0, 0, None)
        acc = accumulate(0, 1, acc)
        for slot in (1, 3, 2):
            for h in range(2):
                rows = pl.ds(h * half, half)
                pltpu.make_async_remote_copy(
                    src_ref=kvbuf.at[slot, :, :, rows, :],
                    dst_ref=kvbuf.at[slot, :, :, rows, :],
                    send_sem=send_sems.at[0, h],
                    recv_sem=recv_sems.at[slot, h],
                    device_id=(my,), device_id_type=pl.DeviceIdType.MESH,
                ).wait_recv()
                acc = accumulate(slot, h, acc)

        loads[4].wait()
        for b in range(B):
            attn_b = jnp.concatenate(
                [(acc[b * HKV + g][j * SQ:(j + 1) * SQ, :DH] /
                  acc[b * HKV + g][j * SQ:(j + 1) * SQ, DH:DH + 1])
                 for g in range(HKV) for j in range(GROUP)], axis=1
            ).astype(jnp.bfloat16)
            out_ref[b] = lax.dot_general(
                attn_b, wo_scr[...], (((1,), (0,)), ((), ())),
                preferred_element_type=jnp.float32,
            )

        for r in sends:
            r.wait_send()

    return pl.pallas_call(
        body,
        out_shape=jax.ShapeDtypeStruct((B, SQ, D), jnp.float32),
        in_specs=[pl.BlockSpec(memory_space=pltpu.HBM)] * 5,
        out_specs=pl.BlockSpec(memory_space=pltpu.VMEM),
        scratch_shapes=[
            pltpu.VMEM((B, SQ, D), jnp.bfloat16),
            pltpu.VMEM((D, D), jnp.bfloat16),
            pltpu.VMEM((D, D), jnp.bfloat16),
            pltpu.VMEM((N_DEV, 2, B, skv_loc, HKV * DH), jnp.bfloat16),
            pltpu.SemaphoreType.DMA((5,)),
            pltpu.SemaphoreType.DMA((N_DEV - 1, 2)),
            pltpu.SemaphoreType.DMA((N_DEV, 2)),
        ],
        compiler_params=pltpu.CompilerParams(collective_id=0),
    )(x2, Wq2, Wo2, K2, V2)


# device time: 14238 ns/iter; 1.0768x vs baseline; 1.0590x over previous
import jax
import jax.numpy as jnp
from jax import lax
from jax.experimental import pallas as pl
from jax.experimental.pallas import tpu as pltpu

N_DEV = 4
B = 2
SQ = 128
D = 512
HQ = 8
HKV = 2
DH = 64
GROUP = HQ // HKV
SCALE = 0.125


def kernel(x, Wq, Wo, K_ext, V_ext):
    skv_loc = K_ext.shape[1]
    K2 = K_ext.reshape(B, skv_loc, HKV * DH).astype(jnp.bfloat16)
    V2 = V_ext.reshape(B, skv_loc, HKV * DH).astype(jnp.bfloat16)
    x2 = x.astype(jnp.bfloat16)
    Wq2 = (Wq * SCALE).astype(jnp.bfloat16)
    Wo2 = Wo.astype(jnp.bfloat16)

    def body(x_hbm, wq_hbm, wo_hbm, k_hbm, v_hbm, out_ref,
             x_scr, wq_scr, wo_scr,
             kvbuf, load_sems, send_sems, recv_sems):
        my = lax.axis_index("i")

        loads = []
        for i, (src, dst) in enumerate(
            [(k_hbm, kvbuf.at[0, 0]), (v_hbm, kvbuf.at[0, 1]),
             (x_hbm, x_scr), (wq_hbm, wq_scr), (wo_hbm, wo_scr)]
        ):
            cp = pltpu.make_async_copy(src, dst, load_sems.at[i])
            cp.start()
            loads.append(cp)

        barrier_sem = pltpu.get_barrier_semaphore()
        for d in range(1, N_DEV):
            pl.semaphore_signal(
                barrier_sem, inc=1,
                device_id=((my + d) % N_DEV,),
                device_id_type=pl.DeviceIdType.MESH,
            )
        loads[0].wait()
        loads[1].wait()
        pl.semaphore_wait(barrier_sem, N_DEV - 1)

        sends = []
        for d in range(1, N_DEV):
            r = pltpu.make_async_remote_copy(
                src_ref=kvbuf.at[0], dst_ref=kvbuf.at[N_DEV - d],
                send_sem=send_sems.at[d - 1], recv_sem=recv_sems.at[N_DEV - d],
                device_id=((my + d) % N_DEV,),
                device_id_type=pl.DeviceIdType.MESH,
            )
            r.start()
            sends.append(r)

        loads[2].wait()
        loads[3].wait()
        wq = wq_scr[...]
        q_stacks = []
        for b in range(B):
            q_b = lax.dot_general(
                x_scr[b], wq, (((1,), (0,)), ((), ())),
                preferred_element_type=jnp.float32,
            ).astype(jnp.bfloat16)
            for g in range(HKV):
                q_stacks.append(jnp.concatenate(
                    [q_b[:, (g * GROUP + j) * DH:(g * GROUP + j + 1) * DH]
                     for j in range(GROUP)], axis=0))

        ones_blk = jnp.ones((skv_loc, DH), jnp.bfloat16)

        def accumulate(slot, acc):
            out = []
            for b in range(B):
                k_c = kvbuf[slot, 0, b]
                v_c = kvbuf[slot, 1, b]
                for g in range(HKV):
                    k_g = k_c[:, g * DH:(g + 1) * DH]
                    v_aug = jnp.concatenate(
                        [v_c[:, g * DH:(g + 1) * DH], ones_blk], axis=1
                    )
                    p = jnp.exp(lax.dot_general(
                        q_stacks[b * HKV + g], k_g, (((1,), (1,)), ((), ())),
                        preferred_element_type=jnp.float32,
                    ).astype(jnp.bfloat16))
                    o = lax.dot_general(
                        p, v_aug, (((1,), (0,)), ((), ())),
                        preferred_element_type=jnp.float32,
                    )
                    i = b * HKV + g
                    out.append(o if acc is None else acc[i] + o)
            return out

        acc = accumulate(0, None)
        for slot in (1, 3, 2):
            pltpu.make_async_remote_copy(
                src_ref=kvbuf.at[slot], dst_ref=kvbuf.at[slot],
                send_sem=send_sems.at[0], recv_sem=recv_sems.at[slot],
                device_id=(my,), device_id_type=pl.DeviceIdType.MESH,
            ).wait_recv()
            acc = accumulate(slot, acc)

        loads[4].wait()
        for b in range(B):
            attn_b = jnp.concatenate(
                [(acc[b * HKV + g][j * SQ:(j + 1) * SQ, :DH] /
                  acc[b * HKV + g][j * SQ:(j + 1) * SQ, DH:DH + 1])
                 for g in range(HKV) for j in range(GROUP)], axis=1
            ).astype(jnp.bfloat16)
            out_ref[b] = lax.dot_general(
                attn_b, wo_scr[...], (((1,), (0,)), ((), ())),
                preferred_element_type=jnp.float32,
            )

        for r in sends:
            r.wait_send()

    return pl.pallas_call(
        body,
        out_shape=jax.ShapeDtypeStruct((B, SQ, D), jnp.float32),
        in_specs=[pl.BlockSpec(memory_space=pltpu.HBM)] * 5,
        out_specs=pl.BlockSpec(memory_space=pltpu.VMEM),
        scratch_shapes=[
            pltpu.VMEM((B, SQ, D), jnp.bfloat16),
            pltpu.VMEM((D, D), jnp.bfloat16),
            pltpu.VMEM((D, D), jnp.bfloat16),
            pltpu.VMEM((N_DEV, 2, B, skv_loc, HKV * DH), jnp.bfloat16),
            pltpu.SemaphoreType.DMA((5,)),
            pltpu.SemaphoreType.DMA((N_DEV - 1,)),
            pltpu.SemaphoreType.DMA((N_DEV,)),
        ],
        compiler_params=pltpu.CompilerParams(collective_id=0),
    )(x2, Wq2, Wo2, K2, V2)
